# baseline (speedup 1.0000x reference)
import functools

import jax
import jax.numpy as jnp
from jax.experimental import pallas as pl
from jax.experimental.pallas import tpu as pltpu
from jax.experimental.pallas import tpu_sc as plsc

N_EMBED = 8192
HALF = N_EMBED // 2
DIM = 32
BETA = 0.25
N_TOKENS = 16 * 1024
BLK_T = 2048
N_BLKS = N_TOKENS // BLK_T

NUM_CORES = 2
NUM_SUBCORES = 16
NUM_WORKERS = NUM_CORES * NUM_SUBCORES
BPW = N_TOKENS // NUM_WORKERS
GATHER_W = 128


def _vq_tc_kernel(lat_ref, emb_ref, l2_ref, e2_ref, ind_ref, loss_ref,
                  sse_ref):
    step = pl.program_id(0)

    @pl.when(step == 0)
    def _init():
        sse_ref[...] = jnp.zeros_like(sse_ref)

    lat = lat_ref[...]
    emb = emb_ref[...]

    lat16 = (2.0 * lat).astype(jnp.bfloat16)
    l2 = l2_ref[...]
    Q = HALF // 2

    def _qreduce(q):
        conv = jax.lax.dot_general(
            lat16, emb[q * Q:(q + 1) * Q], (((1,), (1,)), ((), ())),
            preferred_element_type=jnp.float32)
        n = (conv - l2) - e2_ref[:, q * Q:(q + 1) * Q]
        return jnp.max(n, axis=1), jnp.argmax(n, axis=1) + q * Q

    va, ja = _qreduce(0)
    vb, jb = _qreduce(1)
    vc, jc = _qreduce(2)
    vd, jd = _qreduce(3)
    tb = vb > va
    v0 = jnp.where(tb, vb, va)
    j0 = jnp.where(tb, jb, ja)
    td = vd > vc
    v1 = jnp.where(td, vd, vc)
    j1 = jnp.where(td, jd, jc)
    v0r = v0.astype(jnp.bfloat16).astype(jnp.float32)
    take1 = v1 > v0r
    idx = jnp.where(take1, j1, j0)
    vsel = jnp.where(take1, v1, v0)

    sse_ref[...] += jnp.sum(-vsel).reshape(1, 1)
    ind_ref[0, 0, :] = idx

    @pl.when(step == N_BLKS - 1)
    def _finish():
        mean1 = sse_ref[...] / jnp.float32(N_TOKENS * DIM)
        loss_ref[...] = mean1 + jnp.float32(BETA) * mean1


_sc_mesh = plsc.VectorSubcoreMesh(core_axis_name="c", subcore_axis_name="s")


@functools.partial(
    pl.kernel,
    mesh=_sc_mesh,
    out_type=[
        jax.ShapeDtypeStruct((N_TOKENS, GATHER_W), jnp.float32),
        jax.ShapeDtypeStruct((NUM_CORES, N_EMBED), jnp.float32),
    ],
    scratch_types=[
        pltpu.VMEM((BPW,), jnp.int32),
        pltpu.VMEM((BPW, GATHER_W), jnp.float32),
        pltpu.VMEM((BPW,), jnp.float32),
        pltpu.VMEM_SHARED((N_EMBED,), jnp.float32),
        pltpu.SemaphoreType.DMA,
    ],
)
def _sc_gather_hist(emb_hbm, idx_hbm, zeros_hbm, ones_hbm, out_hbm,
                    counts_hbm, idx_v, rows_v, ones_v, shared, sem):
    cid = jax.lax.axis_index("c")
    sid = jax.lax.axis_index("s")
    wid = sid * NUM_CORES + cid
    base = wid * BPW

    @pl.when(sid == 0)
    def _init():
        pltpu.sync_copy(zeros_hbm, shared)

    pltpu.sync_copy(idx_hbm.at[pl.ds(base, BPW)], idx_v)
    pltpu.sync_copy(ones_hbm, ones_v)
    pltpu.async_copy(emb_hbm.at[idx_v], rows_v, sem).wait()
    pltpu.sync_copy(rows_v, out_hbm.at[pl.ds(base, BPW)])
    plsc.subcore_barrier()
    pltpu.sync_copy(ones_v, shared.at[idx_v], add=True)
    plsc.subcore_barrier()

    @pl.when(sid == 0)
    def _writeback():
        pltpu.sync_copy(shared, counts_hbm.at[cid])


def _unused_tc_kernel(counts_ref, unused_ref):
    total = counts_ref[0, :] + counts_ref[1, :]
    unused_ref[...] = jnp.sum((total == 0.0).astype(jnp.int32)).reshape(1, 1)


@jax.jit
def kernel(x, embed_weight):
    latent = x.reshape(N_TOKENS, DIM)
    l2 = (latent ** 2).sum(axis=1, keepdims=True)
    e2 = (embed_weight ** 2).sum(axis=1, keepdims=True).T
    ind, loss = pl.pallas_call(
        _vq_tc_kernel,
        grid=(N_BLKS,),
        in_specs=[
            pl.BlockSpec((BLK_T, DIM), lambda i: (i, 0)),
            pl.BlockSpec((N_EMBED, DIM), lambda i: (0, 0)),
            pl.BlockSpec((BLK_T, 1), lambda i: (i, 0)),
            pl.BlockSpec((1, N_EMBED), lambda i: (0, 0)),
        ],
        out_specs=[
            pl.BlockSpec((1, 1, BLK_T), lambda i: (i, 0, 0)),
            pl.BlockSpec((1, 1), lambda i: (0, 0)),
        ],
        out_shape=[
            jax.ShapeDtypeStruct((N_BLKS, 1, BLK_T), jnp.int32),
            jax.ShapeDtypeStruct((1, 1), jnp.float32),
        ],
        scratch_shapes=[
            pltpu.VMEM((1, 1), jnp.float32),
        ],
    )(latent, embed_weight, l2, e2)
    idx_flat = ind.reshape(N_TOKENS)
    emb_pad = jnp.pad(embed_weight, ((0, 0), (0, GATHER_W - DIM)))
    zeros = jnp.zeros((N_EMBED,), jnp.float32)
    ones = jnp.ones((BPW,), jnp.float32)
    xq, counts = _sc_gather_hist(emb_pad, idx_flat, zeros, ones)
    unused = pl.pallas_call(
        _unused_tc_kernel,
        out_shape=jax.ShapeDtypeStruct((1, 1), jnp.int32),
    )(counts)
    x_q_st = xq[:, :DIM].reshape(x.shape)
    embed_ind = idx_flat.reshape(x.shape[:-1])
    return (x_q_st, loss.reshape(()), unused.reshape(()), embed_ind)


# speedup vs baseline: 1.1266x; 1.1266x over previous
import functools

import jax
import jax.numpy as jnp
from jax.experimental import pallas as pl
from jax.experimental.pallas import tpu as pltpu
from jax.experimental.pallas import tpu_sc as plsc

N_EMBED = 8192
HALF = N_EMBED // 2
DIM = 32
BETA = 0.25
N_TOKENS = 16 * 1024
BLK_T = 1024
N_BLKS = N_TOKENS // BLK_T

NUM_CORES = 2
NUM_SUBCORES = 16
NUM_WORKERS = NUM_CORES * NUM_SUBCORES
BPW = N_TOKENS // NUM_WORKERS
GATHER_W = 128


def _vq_tc_kernel(lat_ref, emb_ref, l2_ref, e2_ref, ind_ref, loss_ref,
                  sse_ref):
    step = pl.program_id(0)

    @pl.when(step == 0)
    def _init():
        sse_ref[...] = jnp.zeros_like(sse_ref)

    lat = lat_ref[...]
    emb = emb_ref[...]

    lat16 = (2.0 * lat).astype(jnp.bfloat16)
    l2 = l2_ref[...]
    conv0 = jax.lax.dot_general(
        lat16, emb[:HALF], (((1,), (1,)), ((), ())),
        preferred_element_type=jnp.float32)
    n0 = (conv0 - l2) - e2_ref[:, :HALF]
    v0 = jnp.max(n0, axis=1)
    j0 = jnp.argmax(n0, axis=1)
    conv1 = jax.lax.dot_general(
        lat16, emb[HALF:], (((1,), (1,)), ((), ())),
        preferred_element_type=jnp.float32)
    n1 = (conv1 - l2) - e2_ref[:, HALF:]
    v1 = jnp.max(n1, axis=1)
    j1 = jnp.argmax(n1, axis=1) + HALF
    v0r = v0.astype(jnp.bfloat16).astype(jnp.float32)
    take1 = v1 > v0r
    idx = jnp.where(take1, j1, j0)
    vsel = jnp.where(take1, v1, v0)

    sse_ref[...] += jnp.sum(-vsel).reshape(1, 1)
    ind_ref[0, 0, :] = idx

    @pl.when(step == N_BLKS - 1)
    def _finish():
        mean1 = sse_ref[...] / jnp.float32(N_TOKENS * DIM)
        loss_ref[...] = mean1 + jnp.float32(BETA) * mean1


_sc_mesh = plsc.VectorSubcoreMesh(core_axis_name="c", subcore_axis_name="s")


@functools.partial(
    pl.kernel,
    mesh=_sc_mesh,
    out_type=[
        jax.ShapeDtypeStruct((N_TOKENS, GATHER_W), jnp.float32),
        jax.ShapeDtypeStruct((NUM_CORES, N_EMBED), jnp.float32),
    ],
    scratch_types=[
        pltpu.VMEM((BPW,), jnp.int32),
        pltpu.VMEM((BPW, GATHER_W), jnp.float32),
        pltpu.VMEM((BPW,), jnp.float32),
        pltpu.VMEM_SHARED((N_EMBED,), jnp.float32),
        pltpu.SemaphoreType.DMA,
    ],
)
def _sc_gather_hist(emb_hbm, idx_hbm, zeros_hbm, ones_hbm, out_hbm,
                    counts_hbm, idx_v, rows_v, ones_v, shared, sem):
    cid = jax.lax.axis_index("c")
    sid = jax.lax.axis_index("s")
    wid = sid * NUM_CORES + cid
    base = wid * BPW

    @pl.when(sid == 0)
    def _init():
        pltpu.sync_copy(zeros_hbm, shared)

    pltpu.sync_copy(idx_hbm.at[pl.ds(base, BPW)], idx_v)
    pltpu.sync_copy(ones_hbm, ones_v)
    pltpu.async_copy(emb_hbm.at[idx_v], rows_v, sem).wait()
    pltpu.sync_copy(rows_v, out_hbm.at[pl.ds(base, BPW)])
    plsc.subcore_barrier()
    pltpu.sync_copy(ones_v, shared.at[idx_v], add=True)
    plsc.subcore_barrier()

    @pl.when(sid == 0)
    def _writeback():
        pltpu.sync_copy(shared, counts_hbm.at[cid])


def _unused_tc_kernel(counts_ref, unused_ref):
    total = counts_ref[0, :] + counts_ref[1, :]
    unused_ref[...] = jnp.sum((total == 0.0).astype(jnp.int32)).reshape(1, 1)


@jax.jit
def kernel(x, embed_weight):
    latent = x.reshape(N_TOKENS, DIM)
    l2 = (latent ** 2).sum(axis=1, keepdims=True)
    e2 = (embed_weight ** 2).sum(axis=1, keepdims=True).T
    ind, loss = pl.pallas_call(
        _vq_tc_kernel,
        grid=(N_BLKS,),
        in_specs=[
            pl.BlockSpec((BLK_T, DIM), lambda i: (i, 0)),
            pl.BlockSpec((N_EMBED, DIM), lambda i: (0, 0)),
            pl.BlockSpec((BLK_T, 1), lambda i: (i, 0)),
            pl.BlockSpec((1, N_EMBED), lambda i: (0, 0)),
        ],
        out_specs=[
            pl.BlockSpec((1, 1, BLK_T), lambda i: (i, 0, 0)),
            pl.BlockSpec((1, 1), lambda i: (0, 0)),
        ],
        out_shape=[
            jax.ShapeDtypeStruct((N_BLKS, 1, BLK_T), jnp.int32),
            jax.ShapeDtypeStruct((1, 1), jnp.float32),
        ],
        scratch_shapes=[
            pltpu.VMEM((1, 1), jnp.float32),
        ],
    )(latent, embed_weight, l2, e2)
    idx_flat = ind.reshape(N_TOKENS)
    emb_pad = jnp.pad(embed_weight, ((0, 0), (0, GATHER_W - DIM)))
    zeros = jnp.zeros((N_EMBED,), jnp.float32)
    ones = jnp.ones((BPW,), jnp.float32)
    xq, counts = _sc_gather_hist(emb_pad, idx_flat, zeros, ones)
    unused = pl.pallas_call(
        _unused_tc_kernel,
        out_shape=jax.ShapeDtypeStruct((1, 1), jnp.int32),
    )(counts)
    x_q_st = xq[:, :DIM].reshape(x.shape)
    embed_ind = idx_flat.reshape(x.shape[:-1])
    return (x_q_st, loss.reshape(()), unused.reshape(()), embed_ind)
